# baseline (speedup 1.0000x reference)
import functools

import jax
import jax.numpy as jnp
from jax import lax
from jax.experimental import pallas as pl
from jax.experimental.pallas import tpu as pltpu
from jax.experimental.pallas import tpu_sc as plsc

EMB_DIM = 32
LANES = 16


def _mf_kernel_body(bpw, nc,
                    user_hbm, item_hbm, utT_hbm, itT_hbm, ub_hbm, ib_hbm,
                    out_hbm,
                    uidx_v, iidx_v, urows_v, irows_v, ub_v, ib_v, out_v, sem):
    wid = lax.axis_index("s") * nc + lax.axis_index("c")
    base = wid * bpw

    pltpu.sync_copy(user_hbm.at[pl.ds(base, bpw)], uidx_v)
    pltpu.sync_copy(item_hbm.at[pl.ds(base, bpw)], iidx_v)

    copies = []
    for d in range(EMB_DIM):
        copies.append(pltpu.async_copy(
            utT_hbm.at[d].at[uidx_v], urows_v.at[pl.ds(d * bpw, bpw)], sem))
        copies.append(pltpu.async_copy(
            itT_hbm.at[d].at[iidx_v], irows_v.at[pl.ds(d * bpw, bpw)], sem))
    copies.append(pltpu.async_copy(ub_hbm.at[uidx_v], ub_v, sem))
    copies.append(pltpu.async_copy(ib_hbm.at[iidx_v], ib_v, sem))
    for c in copies:
        c.wait()

    def group(g, carry):
        sl = pl.ds(g * LANES, LANES)
        acc = ub_v[sl] + ib_v[sl]
        for d in range(EMB_DIM):
            u = urows_v[pl.ds(d * bpw + g * LANES, LANES)]
            iv = irows_v[pl.ds(d * bpw + g * LANES, LANES)]
            acc = acc + u * iv
        out_v[sl] = acc
        return carry

    lax.fori_loop(0, bpw // LANES, group, 0)

    pltpu.sync_copy(out_v, out_hbm.at[pl.ds(base, bpw)])


def kernel(user, item, user_table, item_table, user_bias_table, item_bias_table):
    batch = user.shape[0]
    info = plsc.get_sparse_core_info()
    nc, ns = info.num_cores, info.num_subcores
    nw = nc * ns
    bpw = batch // nw

    mesh = plsc.VectorSubcoreMesh(core_axis_name="c", subcore_axis_name="s")
    k = pl.kernel(
        functools.partial(_mf_kernel_body, bpw, nc),
        out_type=jax.ShapeDtypeStruct((batch,), jnp.float32),
        mesh=mesh,
        compiler_params=pltpu.CompilerParams(
            use_tc_tiling_on_sc=False, needs_layout_passes=False),
        scratch_types=[
            pltpu.VMEM((bpw,), jnp.int32),
            pltpu.VMEM((bpw,), jnp.int32),
            pltpu.VMEM((bpw * EMB_DIM,), jnp.float32),
            pltpu.VMEM((bpw * EMB_DIM,), jnp.float32),
            pltpu.VMEM((bpw,), jnp.float32),
            pltpu.VMEM((bpw,), jnp.float32),
            pltpu.VMEM((bpw,), jnp.float32),
            pltpu.SemaphoreType.DMA,
        ],
    )
    return k(user.astype(jnp.int32), item.astype(jnp.int32),
             user_table.T, item_table.T,
             user_bias_table.reshape(-1), item_bias_table.reshape(-1))


# speedup vs baseline: 18.7658x; 18.7658x over previous
import functools

import jax
import jax.numpy as jnp
from jax import lax
from jax.experimental import pallas as pl
from jax.experimental.pallas import tpu as pltpu
from jax.experimental.pallas import tpu_sc as plsc

EMB_DIM = 32
LANES = 16
NROWS = 1000000
REG = 999936
TAIL = NROWS - REG
TAIL_BASE = EMB_DIM * REG
CLONE = TAIL_BASE + EMB_DIM * TAIL
CHUNK = 8064
NCHUNK = 31


def _linearize_body(nc, utT_hbm, itT_hbm, ut_tail_hbm, it_tail_hbm,
                    uL_hbm, iL_hbm, buf0, buf1, isem0, isem1, osem0, osem1):
    wid = lax.axis_index("s") * nc + lax.axis_index("c")
    i = wid % 4
    t = (wid // 4) % 2
    q = wid // 8
    row0 = pl.multiple_of(i * 8, 8)
    qcol = pl.multiple_of(q * (NCHUNK * CHUNK), 128)

    bufs = (buf0, buf1)
    isems = (isem0, isem1)
    osems = (osem0, osem1)

    def run(src_hbm, dst_hbm):
        for c in range(NCHUNK):
            b = c % 2
            col = pl.multiple_of(qcol + c * CHUNK, 128)
            if c >= 2:
                pltpu.make_async_copy(
                    src_hbm.at[pl.ds(0, 8), pl.ds(0, CHUNK)],
                    bufs[b], osems[b]).wait()
            pltpu.async_copy(
                src_hbm.at[pl.ds(row0, 8), pl.ds(col, CHUNK)],
                bufs[b], isems[b]).wait()
            for k in range(8):
                d = row0 + k
                dst_off = pl.multiple_of(d * REG + col, 8)
                pltpu.async_copy(bufs[b].at[k, pl.ds(0, CHUNK)],
                                 dst_hbm.at[pl.ds(dst_off, CHUNK)], osems[b])
        for b in range(2):
            pltpu.make_async_copy(
                src_hbm.at[pl.ds(0, 8), pl.ds(0, CHUNK)],
                bufs[b], osems[b]).wait()

    @pl.when(t == 0)
    def _():
        run(utT_hbm, uL_hbm)

    @pl.when(t == 1)
    def _():
        run(itT_hbm, iL_hbm)

    @pl.when(wid == 0)
    def _():
        pltpu.async_copy(ut_tail_hbm, buf0.at[0, pl.ds(0, EMB_DIM * TAIL)],
                         isem0).wait()
        pltpu.async_copy(buf0.at[0, pl.ds(0, EMB_DIM * TAIL)],
                         uL_hbm.at[pl.ds(TAIL_BASE, EMB_DIM * TAIL)],
                         osem0).wait()

    @pl.when(wid == 1)
    def _():
        pltpu.async_copy(it_tail_hbm, buf0.at[0, pl.ds(0, EMB_DIM * TAIL)],
                         isem0).wait()
        pltpu.async_copy(buf0.at[0, pl.ds(0, EMB_DIM * TAIL)],
                         iL_hbm.at[pl.ds(TAIL_BASE, EMB_DIM * TAIL)],
                         osem0).wait()


def _gather_body(bpw, nc,
                 user_hbm, item_hbm, uL_hbm, iL_hbm, ub_hbm, ib_hbm,
                 out_hbm,
                 uidx_v, iidx_v, uim_v, iim_v,
                 urows_v, irows_v, ub_v, ib_v, out_v, sem):
    wid = lax.axis_index("s") * nc + lax.axis_index("c")
    base = wid * bpw

    pltpu.sync_copy(user_hbm.at[pl.ds(base, bpw)], uidx_v)
    pltpu.sync_copy(item_hbm.at[pl.ds(base, bpw)], iidx_v)

    cub = pltpu.async_copy(ub_hbm.at[uidx_v], ub_v, sem)
    cib = pltpu.async_copy(ib_hbm.at[iidx_v], ib_v, sem)

    def build(g, carry):
        sl = pl.ds(g * LANES, LANES)
        for idx_v, im_v in ((uidx_v, uim_v), (iidx_v, iim_v)):
            r = idx_v[sl]
            tail = r >= REG
            main_w = r
            tail_w = (TAIL_BASE - REG) + r
            for d in range(EMB_DIM):
                w = jnp.where(tail, tail_w + d * TAIL, main_w + d * REG)
                im_v[pl.ds(d * bpw + g * LANES, LANES)] = w
        return carry

    lax.fori_loop(0, bpw // LANES, build, 0)

    copies = [cub, cib]
    for d in range(EMB_DIM):
        copies.append(pltpu.async_copy(
            uL_hbm.at[uim_v.at[pl.ds(d * bpw, bpw)]],
            urows_v.at[pl.ds(d * bpw, bpw)], sem))
        copies.append(pltpu.async_copy(
            iL_hbm.at[iim_v.at[pl.ds(d * bpw, bpw)]],
            irows_v.at[pl.ds(d * bpw, bpw)], sem))
    for c in copies:
        c.wait()

    def group(g, carry):
        sl = pl.ds(g * LANES, LANES)
        acc = ub_v[sl] + ib_v[sl]
        for d in range(EMB_DIM):
            u = urows_v[pl.ds(d * bpw + g * LANES, LANES)]
            iv = irows_v[pl.ds(d * bpw + g * LANES, LANES)]
            acc = acc + u * iv
        out_v[sl] = acc
        return carry

    lax.fori_loop(0, bpw // LANES, group, 0)

    pltpu.sync_copy(out_v, out_hbm.at[pl.ds(base, bpw)])


def kernel(user, item, user_table, item_table, user_bias_table, item_bias_table):
    batch = user.shape[0]
    info = plsc.get_sparse_core_info()
    nc, ns = info.num_cores, info.num_subcores
    nw = nc * ns
    bpw = batch // nw

    mesh = plsc.VectorSubcoreMesh(core_axis_name="c", subcore_axis_name="s")

    linearize = pl.kernel(
        functools.partial(_linearize_body, nc),
        out_type=(jax.ShapeDtypeStruct((CLONE,), jnp.float32),
                  jax.ShapeDtypeStruct((CLONE,), jnp.float32)),
        mesh=mesh,
        compiler_params=pltpu.CompilerParams(
            use_tc_tiling_on_sc=True, needs_layout_passes=False),
        scratch_types=[
            pltpu.VMEM((8, CHUNK), jnp.float32),
            pltpu.VMEM((8, CHUNK), jnp.float32),
            pltpu.SemaphoreType.DMA,
            pltpu.SemaphoreType.DMA,
            pltpu.SemaphoreType.DMA,
            pltpu.SemaphoreType.DMA,
        ],
    )
    ut_tail = user_table[REG:].T.reshape(-1)
    it_tail = item_table[REG:].T.reshape(-1)
    uL, iL = linearize(user_table.T, item_table.T, ut_tail, it_tail)

    gather = pl.kernel(
        functools.partial(_gather_body, bpw, nc),
        out_type=jax.ShapeDtypeStruct((batch,), jnp.float32),
        mesh=mesh,
        compiler_params=pltpu.CompilerParams(
            use_tc_tiling_on_sc=False, needs_layout_passes=False),
        scratch_types=[
            pltpu.VMEM((bpw,), jnp.int32),
            pltpu.VMEM((bpw,), jnp.int32),
            pltpu.VMEM((bpw * EMB_DIM,), jnp.int32),
            pltpu.VMEM((bpw * EMB_DIM,), jnp.int32),
            pltpu.VMEM((bpw * EMB_DIM,), jnp.float32),
            pltpu.VMEM((bpw * EMB_DIM,), jnp.float32),
            pltpu.VMEM((bpw,), jnp.float32),
            pltpu.VMEM((bpw,), jnp.float32),
            pltpu.VMEM((bpw,), jnp.float32),
            pltpu.SemaphoreType.DMA,
        ],
    )
    return gather(user.astype(jnp.int32), item.astype(jnp.int32),
                  uL, iL,
                  user_bias_table.reshape(-1), item_bias_table.reshape(-1))
